# Initial kernel scaffold; baseline (speedup 1.0000x reference)
import functools

import jax
import jax.numpy as jnp
from jax import lax
from jax.experimental import pallas as pl
from jax.experimental.pallas import tpu as pltpu
from jax.experimental.pallas import tpu_sc as plsc

NC = 2
NS = 16
NW = NC * NS
BLK = 128

f32 = jnp.float32
i32 = jnp.int32


def _mesh():
    return plsc.VectorSubcoreMesh(
        core_axis_name="c", subcore_axis_name="s", num_cores=NC, num_subcores=NS
    )


def _make_hist(n_pad, nb):
    rpt = n_pad // NS

    @functools.partial(
        pl.kernel,
        out_type=[
            jax.ShapeDtypeStruct((NC, n_pad, 16), f32),
            jax.ShapeDtypeStruct((NC, n_pad, 16), f32),
        ],
        mesh=_mesh(),
        scratch_types=[
            pltpu.VMEM((nb, BLK), i32),
            pltpu.VMEM((nb, BLK), i32),
            pltpu.VMEM((BLK, 16), f32),
            pltpu.VMEM_SHARED((n_pad, 16), f32),
            pltpu.VMEM_SHARED((n_pad, 16), f32),
        ],
    )
    def hist(src_hbm, dst_hbm, ones_hbm, zeros_hbm, hs_hbm, hd_hbm,
             src_v, dst_v, ones_v, acc_s, acc_d):
        cid = lax.axis_index("c")
        sid = lax.axis_index("s")
        wid = sid * NC + cid
        base = sid * rpt
        pltpu.sync_copy(src_hbm.at[wid], src_v)
        pltpu.sync_copy(dst_hbm.at[wid], dst_v)
        pltpu.sync_copy(ones_hbm, ones_v)
        pltpu.sync_copy(zeros_hbm, acc_s.at[pl.ds(base, rpt)])
        pltpu.sync_copy(zeros_hbm, acc_d.at[pl.ds(base, rpt)])
        plsc.subcore_barrier()

        @pl.loop(0, nb)
        def _(b):
            pltpu.sync_copy(ones_v, acc_s.at[src_v.at[b]], add=True)
            pltpu.sync_copy(ones_v, acc_d.at[dst_v.at[b]], add=True)

        plsc.subcore_barrier()
        pltpu.sync_copy(acc_s.at[pl.ds(base, rpt)], hs_hbm.at[cid, pl.ds(base, rpt)])
        pltpu.sync_copy(acc_d.at[pl.ds(base, rpt)], hd_hbm.at[cid, pl.ds(base, rpt)])

    return hist


def _make_agg(n_pad, d, nb):
    rpt = n_pad // NS

    @functools.partial(
        pl.kernel,
        out_type=jax.ShapeDtypeStruct((NC, n_pad, d), f32),
        mesh=_mesh(),
        scratch_types=[
            pltpu.VMEM((nb, BLK), i32),
            pltpu.VMEM((nb, BLK), i32),
            pltpu.VMEM((BLK, d), f32),
            pltpu.VMEM_SHARED((n_pad, d), f32),
            pltpu.SemaphoreType.DMA,
        ],
    )
    def agg(src_hbm, dst_hbm, tbl_hbm, zeros_hbm, out_hbm,
            src_v, dst_v, rows_v, acc, sem):
        cid = lax.axis_index("c")
        sid = lax.axis_index("s")
        wid = sid * NC + cid
        base = sid * rpt
        pltpu.sync_copy(src_hbm.at[wid], src_v)
        pltpu.sync_copy(dst_hbm.at[wid], dst_v)
        pltpu.sync_copy(zeros_hbm, acc.at[pl.ds(base, rpt)])
        plsc.subcore_barrier()

        @pl.loop(0, nb)
        def _(b):
            pltpu.async_copy(tbl_hbm.at[src_v.at[b]], rows_v, sem).wait()
            pltpu.sync_copy(rows_v, acc.at[dst_v.at[b]], add=True)

        plsc.subcore_barrier()
        pltpu.sync_copy(acc.at[pl.ds(base, rpt)], out_hbm.at[cid, pl.ds(base, rpt)])

    return agg


def _tc_feat(x_ref, w_ref, hs_ref, o_ref):
    hs = hs_ref[...]
    deg = 1.0 + hs[0, :, 0] + hs[1, :, 0]
    xw = jnp.dot(x_ref[...], w_ref[...], preferred_element_type=f32,
                 precision=lax.Precision.HIGHEST)
    o_ref[...] = xw * lax.rsqrt(deg)[:, None]


def _tc_mid(a_ref, h1_ref, hs_ref, hd_ref, b1_ref, w2_ref, o_ref):
    a = a_ref[...]
    agg = a[0] + a[1] + h1_ref[...]
    hd = hd_ref[...]
    deg_in = 1.0 + hd[0, :, 0] + hd[1, :, 0]
    y = jnp.maximum(agg * lax.rsqrt(deg_in)[:, None] + b1_ref[...], 0.0)
    s = jnp.sum(y * w2_ref[...], axis=1)
    hs = hs_ref[...]
    deg_out = 1.0 + hs[0, :, 0] + hs[1, :, 0]
    h2 = s * lax.rsqrt(deg_out)
    col = lax.broadcasted_iota(i32, o_ref.shape, 1)
    o_ref[...] = jnp.where(col == 0, h2[:, None], 0.0)


def _make_tc_out(n):
    def _tc_out(a2_ref, h2p_ref, hd_ref, b2_ref, o_ref):
        a2 = a2_ref[...]
        s = a2[0, :, 0] + a2[1, :, 0] + h2p_ref[...][:, 0]
        hd = hd_ref[...]
        deg_in = 1.0 + hd[0, :, 0] + hd[1, :, 0]
        o_ref[...] = (s * lax.rsqrt(deg_in))[:n, None] + b2_ref[...]

    return _tc_out


def kernel(in_feat, edge_index, W1, b1, W2, b2):
    n, d_in = in_feat.shape
    d_h = W1.shape[1]
    e = edge_index.shape[1]

    n_pad = pl.cdiv(n, BLK) * BLK
    ept = pl.cdiv(e, NW * BLK) * BLK
    nb = ept // BLK
    e_pad = ept * NW
    trash = n_pad - 1

    src = edge_index[0].astype(i32)
    dst = edge_index[1].astype(i32)
    fill = jnp.full((e_pad - e,), trash, i32)
    src_p = jnp.concatenate([src, fill]).reshape(NW, nb, BLK)
    dst_p = jnp.concatenate([dst, fill]).reshape(NW, nb, BLK)

    x_pad = jnp.pad(in_feat, ((0, n_pad - n), (0, 0)))
    ones16 = jnp.ones((BLK, 16), f32)
    zeros16 = jnp.zeros((n_pad // NS, 16), f32)
    zeros_d = jnp.zeros((n_pad // NS, d_h), f32)

    hs, hd = _make_hist(n_pad, nb)(src_p, dst_p, ones16, zeros16)

    h1 = pl.pallas_call(
        _tc_feat, out_shape=jax.ShapeDtypeStruct((n_pad, d_h), f32),
    )(x_pad, W1, hs)

    agg1 = _make_agg(n_pad, d_h, nb)(src_p, dst_p, h1, zeros_d)

    h2p = pl.pallas_call(
        _tc_mid, out_shape=jax.ShapeDtypeStruct((n_pad, 16), f32),
    )(agg1, h1, hs, hd, b1.reshape(1, d_h), W2.reshape(1, d_h))

    agg2 = _make_agg(n_pad, 16, nb)(src_p, dst_p, h2p, zeros16)

    out = pl.pallas_call(
        _make_tc_out(n), out_shape=jax.ShapeDtypeStruct((n, 1), f32),
    )(agg2, h2p, hd, b2.reshape(1, 1))
    return out


# speedup vs baseline: 8.7074x; 8.7074x over previous
import functools

import jax
import jax.numpy as jnp
from jax import lax
from jax.experimental import pallas as pl
from jax.experimental.pallas import tpu as pltpu
from jax.experimental.pallas import tpu_sc as plsc

NC = 2
NS = 16
NW = NC * NS
BLK = 128

f32 = jnp.float32
i32 = jnp.int32


def _mesh():
    return plsc.VectorSubcoreMesh(
        core_axis_name="c", subcore_axis_name="s", num_cores=NC, num_subcores=NS
    )


_SC_PARAMS = pltpu.CompilerParams(use_tc_tiling_on_sc=False)


def _make_hist(n_pad, nb):
    rpt = n_pad // NS

    @functools.partial(
        pl.kernel,
        out_type=[
            jax.ShapeDtypeStruct((NC, n_pad, 16), f32),
            jax.ShapeDtypeStruct((NC, n_pad, 16), f32),
        ],
        mesh=_mesh(),
        scratch_types=[
            pltpu.VMEM((nb, BLK), i32),
            pltpu.VMEM((nb, BLK), i32),
            pltpu.VMEM((BLK, 16), f32),
            pltpu.VMEM_SHARED((n_pad, 16), f32),
            pltpu.VMEM_SHARED((n_pad, 16), f32),
        ],
        compiler_params=_SC_PARAMS,
    )
    def hist(src_hbm, dst_hbm, ones_hbm, zeros_hbm, hs_hbm, hd_hbm,
             src_v, dst_v, ones_v, acc_s, acc_d):
        cid = lax.axis_index("c")
        sid = lax.axis_index("s")
        wid = sid * NC + cid
        base = sid * rpt
        pltpu.sync_copy(src_hbm.at[wid], src_v)
        pltpu.sync_copy(dst_hbm.at[wid], dst_v)
        pltpu.sync_copy(ones_hbm, ones_v)
        pltpu.sync_copy(zeros_hbm, acc_s.at[pl.ds(base, rpt)])
        pltpu.sync_copy(zeros_hbm, acc_d.at[pl.ds(base, rpt)])
        plsc.subcore_barrier()

        @pl.loop(0, nb)
        def _(b):
            pltpu.sync_copy(ones_v, acc_s.at[src_v.at[b]], add=True)
            pltpu.sync_copy(ones_v, acc_d.at[dst_v.at[b]], add=True)

        plsc.subcore_barrier()
        pltpu.sync_copy(acc_s.at[pl.ds(base, rpt)], hs_hbm.at[cid, pl.ds(base, rpt)])
        pltpu.sync_copy(acc_d.at[pl.ds(base, rpt)], hd_hbm.at[cid, pl.ds(base, rpt)])

    return hist


def _make_agg(n_pad, d, nb):
    rpt = n_pad // NS

    @functools.partial(
        pl.kernel,
        out_type=jax.ShapeDtypeStruct((NC, n_pad, d), f32),
        mesh=_mesh(),
        scratch_types=[
            pltpu.VMEM((nb, BLK), i32),
            pltpu.VMEM((nb, BLK), i32),
            pltpu.VMEM((BLK, d), f32),
            pltpu.VMEM_SHARED((n_pad, d), f32),
            pltpu.SemaphoreType.DMA,
        ],
        compiler_params=_SC_PARAMS,
    )
    def agg(src_hbm, dst_hbm, tbl_hbm, zeros_hbm, out_hbm,
            src_v, dst_v, rows_v, acc, sem):
        cid = lax.axis_index("c")
        sid = lax.axis_index("s")
        wid = sid * NC + cid
        base = sid * rpt
        pltpu.sync_copy(src_hbm.at[wid], src_v)
        pltpu.sync_copy(dst_hbm.at[wid], dst_v)
        pltpu.sync_copy(zeros_hbm, acc.at[pl.ds(base, rpt)])
        plsc.subcore_barrier()

        @pl.loop(0, nb)
        def _(b):
            pltpu.async_copy(tbl_hbm.at[src_v.at[b]], rows_v, sem).wait()
            pltpu.sync_copy(rows_v, acc.at[dst_v.at[b]], add=True)

        plsc.subcore_barrier()
        pltpu.sync_copy(acc.at[pl.ds(base, rpt)], out_hbm.at[cid, pl.ds(base, rpt)])

    return agg


def _tc_feat(x_ref, w_ref, hs_ref, o_ref):
    hs = hs_ref[...]
    deg = 1.0 + hs[0, :, 0] + hs[1, :, 0]
    xw = jnp.dot(x_ref[...], w_ref[...], preferred_element_type=f32,
                 precision=lax.Precision.HIGHEST)
    o_ref[...] = xw * lax.rsqrt(deg)[:, None]


def _tc_mid(a_ref, h1_ref, hs_ref, hd_ref, b1_ref, w2_ref, o_ref):
    a = a_ref[...]
    agg = a[0] + a[1] + h1_ref[...]
    hd = hd_ref[...]
    deg_in = 1.0 + hd[0, :, 0] + hd[1, :, 0]
    y = jnp.maximum(agg * lax.rsqrt(deg_in)[:, None] + b1_ref[...], 0.0)
    s = jnp.sum(y * w2_ref[...], axis=1)
    hs = hs_ref[...]
    deg_out = 1.0 + hs[0, :, 0] + hs[1, :, 0]
    h2 = s * lax.rsqrt(deg_out)
    col = lax.broadcasted_iota(i32, o_ref.shape, 1)
    o_ref[...] = jnp.where(col == 0, h2[:, None], 0.0)


def _make_tc_out(n):
    def _tc_out(a2_ref, h2p_ref, hd_ref, b2_ref, o_ref):
        a2 = a2_ref[...]
        s = a2[0, :, 0] + a2[1, :, 0] + h2p_ref[...][:, 0]
        hd = hd_ref[...]
        deg_in = 1.0 + hd[0, :, 0] + hd[1, :, 0]
        o_ref[...] = (s * lax.rsqrt(deg_in))[:n, None] + b2_ref[...]

    return _tc_out


def kernel(in_feat, edge_index, W1, b1, W2, b2):
    n, d_in = in_feat.shape
    d_h = W1.shape[1]
    e = edge_index.shape[1]

    n_pad = pl.cdiv(n, BLK) * BLK
    ept = pl.cdiv(e, NW * BLK) * BLK
    nb = ept // BLK
    e_pad = ept * NW
    trash = n_pad - 1

    src = edge_index[0].astype(i32)
    dst = edge_index[1].astype(i32)
    fill = jnp.full((e_pad - e,), trash, i32)
    src_p = jnp.concatenate([src, fill]).reshape(NW, nb, BLK)
    dst_p = jnp.concatenate([dst, fill]).reshape(NW, nb, BLK)

    x_pad = jnp.pad(in_feat, ((0, n_pad - n), (0, 0)))
    ones16 = jnp.ones((BLK, 16), f32)
    zeros16 = jnp.zeros((n_pad // NS, 16), f32)
    zeros_d = jnp.zeros((n_pad // NS, d_h), f32)

    hs, hd = _make_hist(n_pad, nb)(src_p, dst_p, ones16, zeros16)

    h1 = pl.pallas_call(
        _tc_feat, out_shape=jax.ShapeDtypeStruct((n_pad, d_h), f32),
    )(x_pad, W1, hs)

    agg1 = _make_agg(n_pad, d_h, nb)(src_p, dst_p, h1, zeros_d)

    h2p = pl.pallas_call(
        _tc_mid, out_shape=jax.ShapeDtypeStruct((n_pad, 16), f32),
    )(agg1, h1, hs, hd, b1.reshape(1, d_h), W2.reshape(1, d_h))

    agg2 = _make_agg(n_pad, 16, nb)(src_p, dst_p, h2p, zeros16)

    out = pl.pallas_call(
        _make_tc_out(n), out_shape=jax.ShapeDtypeStruct((n, 1), f32),
    )(agg2, h2p, hd, b2.reshape(1, 1))
    return out


# speedup vs baseline: 10.6521x; 1.2233x over previous
import functools

import jax
import jax.numpy as jnp
from jax import lax
from jax.experimental import pallas as pl
from jax.experimental.pallas import tpu as pltpu
from jax.experimental.pallas import tpu_sc as plsc

NC = 2
NS = 16
NW = NC * NS
BLK = 128

f32 = jnp.float32
i32 = jnp.int32


def _mesh():
    return plsc.VectorSubcoreMesh(
        core_axis_name="c", subcore_axis_name="s", num_cores=NC, num_subcores=NS
    )


_SC_PARAMS = pltpu.CompilerParams(use_tc_tiling_on_sc=False)


def _make_hist(n_pad, nb):
    rpt = n_pad // NS

    @functools.partial(
        pl.kernel,
        out_type=[
            jax.ShapeDtypeStruct((NC, n_pad, 16), f32),
            jax.ShapeDtypeStruct((NC, n_pad, 16), f32),
        ],
        mesh=_mesh(),
        scratch_types=[
            pltpu.VMEM((nb, BLK), i32),
            pltpu.VMEM((nb, BLK), i32),
            pltpu.VMEM((BLK, 16), f32),
            pltpu.VMEM_SHARED((n_pad, 16), f32),
            pltpu.VMEM_SHARED((n_pad, 16), f32),
        ],
        compiler_params=_SC_PARAMS,
    )
    def hist(src_hbm, dst_hbm, ones_hbm, zeros_hbm, hs_hbm, hd_hbm,
             src_v, dst_v, ones_v, acc_s, acc_d):
        cid = lax.axis_index("c")
        sid = lax.axis_index("s")
        wid = sid * NC + cid
        base = sid * rpt
        pltpu.sync_copy(src_hbm.at[wid], src_v)
        pltpu.sync_copy(dst_hbm.at[wid], dst_v)
        pltpu.sync_copy(ones_hbm, ones_v)
        pltpu.sync_copy(zeros_hbm, acc_s.at[pl.ds(base, rpt)])
        pltpu.sync_copy(zeros_hbm, acc_d.at[pl.ds(base, rpt)])
        plsc.subcore_barrier()

        @pl.loop(0, nb)
        def _(b):
            pltpu.sync_copy(ones_v, acc_s.at[src_v.at[b]], add=True)
            pltpu.sync_copy(ones_v, acc_d.at[dst_v.at[b]], add=True)

        plsc.subcore_barrier()
        pltpu.sync_copy(acc_s.at[pl.ds(base, rpt)], hs_hbm.at[cid, pl.ds(base, rpt)])
        pltpu.sync_copy(acc_d.at[pl.ds(base, rpt)], hd_hbm.at[cid, pl.ds(base, rpt)])

    return hist


NBUF = 4


def _ring_loop(nb, tbl_hbm, src_v, dst_v, rows_v, acc, sems):
    for j in range(NBUF):
        pltpu.async_copy(tbl_hbm.at[src_v.at[j]], rows_v.at[j], sems[j])

    @pl.loop(0, nb, step=NBUF)
    def _(b):
        for j in range(NBUF):
            pltpu.make_async_copy(
                tbl_hbm.at[src_v.at[b + j]], rows_v.at[j], sems[j]
            ).wait()
            pltpu.sync_copy(rows_v.at[j], acc.at[dst_v.at[b + j]], add=True)
            nxt = b + j + NBUF

            @pl.when(nxt < nb)
            def _():
                pltpu.async_copy(tbl_hbm.at[src_v.at[nxt]], rows_v.at[j],
                                 sems[j])


def _make_agg_feat(n_pad, d, nb):
    rpt = n_pad // NS

    @functools.partial(
        pl.kernel,
        out_type=jax.ShapeDtypeStruct((NC, n_pad, d), f32),
        mesh=_mesh(),
        scratch_types=[
            pltpu.VMEM((nb, BLK), i32),
            pltpu.VMEM((nb, BLK), i32),
            pltpu.VMEM((NBUF, BLK, d), f32),
            pltpu.VMEM_SHARED((n_pad, d), f32),
        ] + [pltpu.SemaphoreType.DMA] * NBUF,
        compiler_params=_SC_PARAMS,
    )
    def agg(src_hbm, dst_hbm, tbl_hbm, zeros_hbm, out_hbm,
            src_v, dst_v, rows_v, acc, *sems):
        cid = lax.axis_index("c")
        sid = lax.axis_index("s")
        base = sid * rpt
        pltpu.sync_copy(src_hbm.at[cid, sid], src_v)
        pltpu.sync_copy(dst_hbm.at[sid], dst_v)
        pltpu.sync_copy(zeros_hbm, acc.at[pl.ds(base, rpt)])
        plsc.subcore_barrier()
        _ring_loop(nb, tbl_hbm, src_v, dst_v, rows_v, acc, sems)
        plsc.subcore_barrier()
        pltpu.sync_copy(acc.at[pl.ds(base, rpt)], out_hbm.at[cid, pl.ds(base, rpt)])

    return agg


def _make_agg_edge(n_pad, d, nb):
    rpt = n_pad // NS

    @functools.partial(
        pl.kernel,
        out_type=jax.ShapeDtypeStruct((NC, n_pad, d), f32),
        mesh=_mesh(),
        scratch_types=[
            pltpu.VMEM((nb, BLK), i32),
            pltpu.VMEM((nb, BLK), i32),
            pltpu.VMEM((NBUF, BLK, d), f32),
            pltpu.VMEM_SHARED((n_pad, d), f32),
        ] + [pltpu.SemaphoreType.DMA] * NBUF,
        compiler_params=_SC_PARAMS,
    )
    def agg(src_hbm, dst_hbm, tbl_hbm, zeros_hbm, out_hbm,
            src_v, dst_v, rows_v, acc, *sems):
        cid = lax.axis_index("c")
        sid = lax.axis_index("s")
        wid = sid * NC + cid
        base = sid * rpt
        pltpu.sync_copy(src_hbm.at[wid], src_v)
        pltpu.sync_copy(dst_hbm.at[wid], dst_v)
        pltpu.sync_copy(zeros_hbm, acc.at[pl.ds(base, rpt)])
        plsc.subcore_barrier()
        _ring_loop(nb, tbl_hbm, src_v, dst_v, rows_v, acc, sems)
        plsc.subcore_barrier()
        pltpu.sync_copy(acc.at[pl.ds(base, rpt)], out_hbm.at[cid, pl.ds(base, rpt)])

    return agg


def _make_tc_feat(dh):
    def _tc_feat(x_ref, w_ref, hs_ref, o_ref):
        hs = hs_ref[...]
        deg = 1.0 + hs[0, :, 0] + hs[1, :, 0]
        xw = jnp.dot(x_ref[...], w_ref[...], preferred_element_type=f32,
                     precision=lax.Precision.HIGHEST)
        h1 = xw * lax.rsqrt(deg)[:, None]
        o_ref[...] = jnp.stack([h1[:, :dh], h1[:, dh:]])

    return _tc_feat


def _make_tc_mid():
    def _tc_mid(a_ref, h1_ref, hs_ref, hd_ref, b1_ref, w2_ref, o_ref):
        a = a_ref[...]
        h1s = h1_ref[...]
        h1 = jnp.concatenate([h1s[0], h1s[1]], axis=1)
        agg = jnp.concatenate([a[0], a[1]], axis=1) + h1
        hd = hd_ref[...]
        deg_in = 1.0 + hd[0, :, 0] + hd[1, :, 0]
        y = jnp.maximum(agg * lax.rsqrt(deg_in)[:, None] + b1_ref[...], 0.0)
        s = jnp.sum(y * w2_ref[...], axis=1)
        hs = hs_ref[...]
        deg_out = 1.0 + hs[0, :, 0] + hs[1, :, 0]
        h2 = s * lax.rsqrt(deg_out)
        col = lax.broadcasted_iota(i32, o_ref.shape, 1)
        o_ref[...] = jnp.where(col == 0, h2[:, None], 0.0)

    return _tc_mid


def _make_tc_out(n):
    def _tc_out(a2_ref, h2p_ref, hd_ref, b2_ref, o_ref):
        a2 = a2_ref[...]
        s = a2[0, :, 0] + a2[1, :, 0] + h2p_ref[...][:, 0]
        hd = hd_ref[...]
        deg_in = 1.0 + hd[0, :, 0] + hd[1, :, 0]
        o_ref[...] = (s * lax.rsqrt(deg_in))[:n, None] + b2_ref[...]

    return _tc_out


def kernel(in_feat, edge_index, W1, b1, W2, b2):
    n, d_in = in_feat.shape
    d_h = W1.shape[1]
    e = edge_index.shape[1]

    n_pad = pl.cdiv(n, BLK) * BLK
    ept = pl.cdiv(e, NW * NBUF * BLK) * NBUF * BLK
    nb = ept // BLK
    e_pad = ept * NW
    trash = n_pad - 1

    nbf = e_pad // (NS * BLK)
    dh = d_h // NC

    src = edge_index[0].astype(i32)
    dst = edge_index[1].astype(i32)
    fill = jnp.full((e_pad - e,), trash, i32)
    src_f = jnp.concatenate([src, fill])
    dst_f = jnp.concatenate([dst, fill])
    src_p = src_f.reshape(NW, nb, BLK)
    dst_p = dst_f.reshape(NW, nb, BLK)
    src16 = src_f.reshape(NS, nbf, BLK)
    src_feat = jnp.stack([src16, src16 + n_pad])
    dst16 = dst_f.reshape(NS, nbf, BLK)

    x_pad = jnp.pad(in_feat, ((0, n_pad - n), (0, 0)))
    ones16 = jnp.ones((BLK, 16), f32)
    zeros16 = jnp.zeros((n_pad // NS, 16), f32)
    zeros_dh = jnp.zeros((n_pad // NS, dh), f32)

    hs, hd = _make_hist(n_pad, nb)(src_p, dst_p, ones16, zeros16)

    RB = n_pad // 8
    h1s = pl.pallas_call(
        _make_tc_feat(dh),
        grid=(n_pad // RB,),
        in_specs=[
            pl.BlockSpec((RB, d_in), lambda i: (i, 0)),
            pl.BlockSpec((d_in, d_h), lambda i: (0, 0)),
            pl.BlockSpec((NC, RB, 16), lambda i: (0, i, 0)),
        ],
        out_specs=pl.BlockSpec((NC, RB, dh), lambda i: (0, i, 0)),
        out_shape=jax.ShapeDtypeStruct((NC, n_pad, dh), f32),
    )(x_pad, W1, hs)

    agg1 = _make_agg_feat(n_pad, dh, nbf)(
        src_feat, dst16, h1s.reshape(NC * n_pad, dh), zeros_dh)

    h2p = pl.pallas_call(
        _make_tc_mid(),
        grid=(n_pad // RB,),
        in_specs=[
            pl.BlockSpec((NC, RB, dh), lambda i: (0, i, 0)),
            pl.BlockSpec((NC, RB, dh), lambda i: (0, i, 0)),
            pl.BlockSpec((NC, RB, 16), lambda i: (0, i, 0)),
            pl.BlockSpec((NC, RB, 16), lambda i: (0, i, 0)),
            pl.BlockSpec((1, d_h), lambda i: (0, 0)),
            pl.BlockSpec((1, d_h), lambda i: (0, 0)),
        ],
        out_specs=pl.BlockSpec((RB, 16), lambda i: (i, 0)),
        out_shape=jax.ShapeDtypeStruct((n_pad, 16), f32),
    )(agg1, h1s, hs, hd, b1.reshape(1, d_h), W2.reshape(1, d_h))

    agg2 = _make_agg_edge(n_pad, 16, nb)(src_p, dst_p, h2p, zeros16)

    out = pl.pallas_call(
        _make_tc_out(n), out_shape=jax.ShapeDtypeStruct((n, 1), f32),
    )(agg2, h2p, hd, b2.reshape(1, 1))
    return out
